# baseline (speedup 1.0000x reference)
import functools

import jax
import jax.numpy as jnp
from jax import lax
from jax.experimental import pallas as pl
from jax.experimental.pallas import tpu as pltpu
from jax.experimental.pallas import tpu_sc as plsc

VOCAB = 1000000
EMBED_DIM = 64
BATCH = 4096
HIST = 200
B_TOTAL = BATCH * HIST

SUB = 128
CHUNK = 512
K = CHUNK // SUB


@functools.cache
def _build_gather():
    info = plsc.get_sparse_core_info()
    nw = info.num_cores * info.num_subcores
    b_per_w = B_TOTAL // nw
    n_chunks = b_per_w // CHUNK
    assert b_per_w % CHUNK == 0

    mesh = plsc.VectorSubcoreMesh(core_axis_name="c", subcore_axis_name="s")

    @functools.partial(
        pl.kernel,
        mesh=mesh,
        out_type=jax.ShapeDtypeStruct((B_TOTAL, EMBED_DIM), jnp.float32),
        scratch_types=[
            pltpu.VMEM((K, SUB), jnp.int32),
            pltpu.VMEM((CHUNK, EMBED_DIM), jnp.float32),
            pltpu.SemaphoreType.DMA,
        ],
        compiler_params=pltpu.CompilerParams(use_tc_tiling_on_sc=False),
    )
    def gather_kernel(idx_hbm, table_hbm, out_hbm, idx_v, rows_v, sem):
        wid = lax.axis_index("s") * info.num_cores + lax.axis_index("c")
        row0 = wid * (b_per_w // SUB)

        def body(i, _):
            r = row0 + i * K
            pltpu.sync_copy(idx_hbm.at[pl.ds(r, K)], idx_v)
            copies = []
            for j in range(K):
                copies.append(
                    pltpu.async_copy(
                        table_hbm.at[idx_v.at[j]],
                        rows_v.at[pl.ds(j * SUB, SUB)],
                        sem,
                    )
                )
            for c in copies:
                c.wait()
            pltpu.sync_copy(rows_v, out_hbm.at[pl.ds(r * SUB, CHUNK)])
            return ()

        lax.fori_loop(0, n_chunks, body, ())

    return gather_kernel


def kernel(x, word_vectors):
    idx2d = x.reshape(B_TOTAL // SUB, SUB)
    out = _build_gather()(idx2d, word_vectors)
    return out.reshape(BATCH, HIST, EMBED_DIM)


# speedup vs baseline: 1.0423x; 1.0423x over previous
import functools

import jax
import jax.numpy as jnp
from jax import lax
from jax.experimental import pallas as pl
from jax.experimental.pallas import tpu as pltpu
from jax.experimental.pallas import tpu_sc as plsc

VOCAB = 1000000
EMBED_DIM = 64
BATCH = 4096
HIST = 200
B_TOTAL = BATCH * HIST

SUB = 128
CHUNK = 256
K = CHUNK // SUB
NBUF = 4
DRAIN_LAG = 2


@functools.cache
def _build_gather():
    info = plsc.get_sparse_core_info()
    nw = info.num_cores * info.num_subcores
    b_per_w = B_TOTAL // nw
    n_chunks = b_per_w // CHUNK
    rows_per_w = b_per_w // SUB
    assert b_per_w % CHUNK == 0 and n_chunks > NBUF

    mesh = plsc.VectorSubcoreMesh(core_axis_name="c", subcore_axis_name="s")

    @functools.partial(
        pl.kernel,
        mesh=mesh,
        out_type=jax.ShapeDtypeStruct((B_TOTAL, EMBED_DIM), jnp.float32),
        scratch_types=[
            pltpu.VMEM((rows_per_w, SUB), jnp.int32),
            pltpu.VMEM((NBUF, CHUNK, EMBED_DIM), jnp.float32),
        ]
        + [pltpu.SemaphoreType.DMA] * (2 * NBUF),
        compiler_params=pltpu.CompilerParams(use_tc_tiling_on_sc=False),
    )
    def gather_kernel(idx_hbm, table_hbm, out_hbm, idx_v, rows_v, *sems):
        gsem = sems[:NBUF]
        wsem = sems[NBUF:]
        wid = lax.axis_index("s") * info.num_cores + lax.axis_index("c")
        row0 = wid * rows_per_w
        base = wid * b_per_w

        pltpu.sync_copy(idx_hbm.at[pl.ds(row0, rows_per_w)], idx_v)

        def fire_gathers(i, b):
            for j in range(K):
                pltpu.async_copy(
                    table_hbm.at[idx_v.at[i * K + j]],
                    rows_v.at[b].at[pl.ds(j * SUB, SUB)],
                    gsem[b],
                )

        def drain_gathers(b):
            pltpu.make_async_copy(
                out_hbm.at[pl.ds(0, CHUNK)], rows_v.at[b], gsem[b]
            ).wait()

        def fire_write(i, b):
            pltpu.async_copy(
                rows_v.at[b], out_hbm.at[pl.ds(base + i * CHUNK, CHUNK)], wsem[b]
            )

        def drain_write(b):
            pltpu.make_async_copy(
                out_hbm.at[pl.ds(0, CHUNK)], rows_v.at[b], wsem[b]
            ).wait()

        for s in range(NBUF):
            fire_gathers(s, s)
            if s >= DRAIN_LAG:
                bb = s - DRAIN_LAG
                drain_gathers(bb)
                fire_write(s - DRAIN_LAG, bb)

        n_steady = n_chunks - NBUF
        assert n_steady % NBUF == 0
        def outer(m, _):
            i0 = NBUF + m * NBUF
            for b in range(NBUF):
                i = i0 + b
                drain_write(b)
                fire_gathers(i, b)
                bb = (b - DRAIN_LAG) % NBUF
                drain_gathers(bb)
                fire_write(i - DRAIN_LAG, bb)
            return ()

        lax.fori_loop(0, n_steady // NBUF, outer, ())

        for d in range(DRAIN_LAG, 0, -1):
            i = n_chunks - d
            b = i % NBUF
            drain_gathers(b)
            fire_write(i, b)
        for b in range(NBUF):
            drain_write(b)

    return gather_kernel


def kernel(x, word_vectors):
    idx2d = x.reshape(B_TOTAL // SUB, SUB)
    out = _build_gather()(idx2d, word_vectors)
    return out.reshape(BATCH, HIST, EMBED_DIM)


# speedup vs baseline: 1.0434x; 1.0010x over previous
import functools

import jax
import jax.numpy as jnp
from jax import lax
from jax.experimental import pallas as pl
from jax.experimental.pallas import tpu as pltpu
from jax.experimental.pallas import tpu_sc as plsc

VOCAB = 1000000
EMBED_DIM = 64
BATCH = 4096
HIST = 200

GATHER_SPLIT = (0, 120)
GATHER_LENS = (120, 80)
NBUF = 4
DRAIN_LAG = 2


@functools.cache
def _build_gather():
    info = plsc.get_sparse_core_info()
    nw = info.num_cores * info.num_subcores
    rows_per_w = BATCH // nw
    assert BATCH % nw == 0 and rows_per_w > NBUF

    mesh = plsc.VectorSubcoreMesh(core_axis_name="c", subcore_axis_name="s")

    @functools.partial(
        pl.kernel,
        mesh=mesh,
        out_type=jax.ShapeDtypeStruct((BATCH, HIST, EMBED_DIM), jnp.float32),
        scratch_types=[
            pltpu.VMEM((rows_per_w, HIST), jnp.int32),
            pltpu.VMEM((NBUF, HIST, EMBED_DIM), jnp.float32),
        ]
        + [pltpu.SemaphoreType.DMA] * (2 * NBUF),
        compiler_params=pltpu.CompilerParams(use_tc_tiling_on_sc=False),
    )
    def gather_kernel(idx_hbm, table_hbm, out_hbm, idx_v, rows_v, *sems):
        gsem = sems[:NBUF]
        wsem = sems[NBUF:]
        wid = lax.axis_index("s") * info.num_cores + lax.axis_index("c")
        base = wid * rows_per_w

        pltpu.sync_copy(idx_hbm.at[pl.ds(base, rows_per_w)], idx_v)

        def fire_gathers(i, b):
            for off, ln in zip(GATHER_SPLIT, GATHER_LENS):
                pltpu.async_copy(
                    table_hbm.at[idx_v.at[i, pl.ds(off, ln)]],
                    rows_v.at[b].at[pl.ds(off, ln)],
                    gsem[b],
                )

        def drain_gathers(b):
            pltpu.make_async_copy(
                out_hbm.at[0], rows_v.at[b], gsem[b]
            ).wait()

        def fire_write(i, b):
            pltpu.async_copy(rows_v.at[b], out_hbm.at[base + i], wsem[b])

        def drain_write(b):
            pltpu.make_async_copy(
                out_hbm.at[0], rows_v.at[b], wsem[b]
            ).wait()

        for s in range(NBUF):
            fire_gathers(s, s)
            if s >= DRAIN_LAG:
                bb = s - DRAIN_LAG
                drain_gathers(bb)
                fire_write(s - DRAIN_LAG, bb)

        n_steady = rows_per_w - NBUF
        assert n_steady % NBUF == 0

        def outer(m, _):
            i0 = NBUF + m * NBUF
            for b in range(NBUF):
                i = i0 + b
                drain_write(b)
                fire_gathers(i, b)
                bb = (b - DRAIN_LAG) % NBUF
                drain_gathers(bb)
                fire_write(i - DRAIN_LAG, bb)
            return ()

        lax.fori_loop(0, n_steady // NBUF, outer, ())

        for d in range(DRAIN_LAG, 0, -1):
            i = rows_per_w - d
            b = i % NBUF
            drain_gathers(b)
            fire_write(i, b)
        for b in range(NBUF):
            drain_write(b)

    return gather_kernel


def kernel(x, word_vectors):
    return _build_gather()(x, word_vectors)
